# baseline (speedup 1.0000x reference)
import functools

import jax
import jax.numpy as jnp
from jax import lax
from jax.experimental import pallas as pl
from jax.experimental.pallas import tpu as pltpu
from jax.experimental.pallas import tpu_sc as plsc

BATCH = 4096
SEQ = 200
EMBED = 64
TOTAL = BATCH * SEQ

NUM_CORES = 2
NUM_SUBCORES = 16
NW = NUM_CORES * NUM_SUBCORES
ROWS_PER_W = TOTAL // NW

CHUNK = 128
NCHUNK = ROWS_PER_W // CHUNK
NBUF = 4
ROUNDS = NCHUNK // NBUF


def _emb_body(seq_hbm, table_hbm, out_hbm, idx_v, bufs, gsem, wsem):
    wid = lax.axis_index("s") * NUM_CORES + lax.axis_index("c")
    base_chunk = wid * NCHUNK

    pltpu.sync_copy(seq_hbm.at[pl.ds(base_chunk, NCHUNK)], idx_v)

    chunk_bytes = CHUNK * EMBED * 4

    def fire_gathers(r):
        par = lax.rem(r, 2)
        for b in range(NBUF):
            c = r * NBUF + b
            pltpu.async_copy(table_hbm.at[idx_v.at[c]], bufs.at[par, b], gsem)

    def fire_writebacks(r):
        par = lax.rem(r, 2)
        for b in range(NBUF):
            c = r * NBUF + b
            pltpu.async_copy(
                bufs.at[par, b],
                out_hbm.at[pl.ds((base_chunk + c) * CHUNK, CHUNK)],
                wsem,
            )

    def drain(sem, n):
        for _ in range(n):
            pltpu.make_async_copy(
                table_hbm.at[pl.ds(0, CHUNK)], bufs.at[0, 0], sem
            ).wait()

    fire_gathers(0)

    def round_body(r, carry):
        drain(gsem, NBUF)

        @pl.when(r >= 1)
        def _():
            drain(wsem, NBUF)

        @pl.when(r + 1 < ROUNDS)
        def _():
            fire_gathers(r + 1)

        fire_writebacks(r)
        return carry

    lax.fori_loop(0, ROUNDS, round_body, 0, unroll=False)
    drain(wsem, NBUF)


@functools.partial(jax.jit, static_argnames=())
def kernel(sequence, table):
    seq2d = sequence.reshape(NW * NCHUNK, CHUNK).astype(jnp.int32)
    mesh = plsc.VectorSubcoreMesh(core_axis_name="c", subcore_axis_name="s")
    run = pl.kernel(
        _emb_body,
        out_type=jax.ShapeDtypeStruct((TOTAL, EMBED), jnp.float32),
        mesh=mesh,
        scratch_types=[
            pltpu.VMEM((NCHUNK, CHUNK), jnp.int32),
            pltpu.VMEM((2, NBUF, CHUNK, EMBED), jnp.float32),
            pltpu.SemaphoreType.DMA,
            pltpu.SemaphoreType.DMA,
        ],
        compiler_params=pltpu.CompilerParams(use_tc_tiling_on_sc=False),
    )
    out = run(seq2d, table)
    return out.reshape(BATCH, SEQ, EMBED)
